# baseline (speedup 1.0000x reference)
import jax
import jax.numpy as jnp
from jax import lax
from jax.experimental import pallas as pl
from jax.experimental.pallas import tpu as pltpu
from jax.experimental.pallas import tpu_sc as plsc

N_NODES = 10000
C = 128
CH = C // 2
N_EDGES = 320000

NC = 2
NS = 16
K = 128
CHUNKS = 6 * (-(-N_EDGES // (NS * K * 6)))
PER_S = CHUNKS * K
E_PAD = PER_S * NS
NP = N_NODES // 2
SLABP = 312
TAILP = NP - NS * SLABP


def _sc_body(xP, meta, wgt, out, xsp, acc, midx, wbuf, rows,
             gsem0, gsem1, msem0, msem1, msem2, wsem0, wsem1, wsem2, ssem):
    cid = lax.axis_index("c")
    sid = lax.axis_index("s")
    gsems = (gsem0, gsem1)
    msems = (msem0, msem1, msem2)
    wsems = (wsem0, wsem1, wsem2)

    def meta_cp(ch, b3):
        return pltpu.make_async_copy(meta.at[sid, ch], midx.at[b3], msems[b3])

    def wgt_cp(ch, b3):
        return pltpu.make_async_copy(wgt.at[sid, ch], wbuf.at[b3], wsems[b3])

    def gather(ch, b2, b3):
        return pltpu.make_async_copy(
            xsp.at[midx.at[b3, 1]], rows.at[b2], gsems[b2])

    def scatter_wait(b2, b3):
        pltpu.make_async_copy(rows.at[b2], acc.at[midx.at[b3, 0]], ssem).wait()

    p0 = pl.multiple_of(sid * SLABP, 8)
    poff = 0
    while poff < SLABP:
        n = min(K, SLABP - poff)
        pltpu.sync_copy(xP.at[cid, pl.ds(p0 + poff, n)],
                        xsp.at[pl.ds(p0 + poff, n)])
        poff += n

    @pl.when(sid == 0)
    def _stage_tail():
        pltpu.sync_copy(xP.at[cid, pl.ds(NS * SLABP, TAILP)],
                        xsp.at[pl.ds(NS * SLABP, TAILP)])

    def zero_row(i, _):
        for j in range(C // 16):
            rows[0, i, pl.ds(16 * j, 16)] = jnp.zeros((16,), jnp.float32)
        return 0
    lax.fori_loop(0, K, zero_row, 0)
    poff = 0
    while poff < SLABP:
        n = min(K, SLABP - poff)
        pltpu.sync_copy(rows.at[0, pl.ds(0, n)], acc.at[pl.ds(p0 + poff, n)])
        poff += n

    @pl.when(sid == 0)
    def _zero_tail():
        pltpu.sync_copy(rows.at[0, pl.ds(0, TAILP)],
                        acc.at[pl.ds(NS * SLABP, TAILP)])
    plsc.subcore_barrier()

    for c0 in range(3):
        meta_cp(c0, c0).start()
        wgt_cp(c0, c0).start()
    meta_cp(0, 0).wait()
    wgt_cp(0, 0).wait()
    gather(0, 0, 0).start()

    zeros16 = jnp.zeros((16,), jnp.float32)

    def six(g, _):
        for q in range(6):
            b2 = q % 2
            b3 = q % 3
            t = 6 * g + q
            gather(t, b2, b3).wait()

            @pl.when(t >= 1)
            def _drain_scatter():
                scatter_wait(1 - b2, (q + 2) % 3)

            @pl.when(jnp.logical_and(t >= 1, t + 2 < CHUNKS))
            def _prefetch_meta():
                meta_cp(t + 2, (q + 2) % 3).start()
                wgt_cp(t + 2, (q + 2) % 3).start()

            def scale(g8, _):
                wv = wbuf[b3, pl.ds(16 * g8, 16)]
                w2v = wv * wv
                iov = midx[b3, 2, pl.ds(16 * g8, 16)]
                for l in range(16):
                    e = 16 * g8 + l
                    w2 = w2v[l]
                    v = iov[l]
                    io = v & 255
                    jo = lax.shift_right_logical(v, 8)
                    oio = CH - io
                    for k in range(CH // 16):
                        rows[b2, e, pl.ds(io + 16 * k, 16)] = (
                            rows[b2, e, pl.ds(jo + 16 * k, 16)] * w2)
                        rows[b2, e, pl.ds(oio + 16 * k, 16)] = zeros16
                return 0
            lax.fori_loop(0, K // 16, scale, 0)

            pltpu.async_copy(rows.at[b2], acc.at[midx.at[b3, 0]], ssem, add=True)

            @pl.when(t + 1 < CHUNKS)
            def _next_gather():
                meta_cp(t + 1, (q + 1) % 3).wait()
                wgt_cp(t + 1, (q + 1) % 3).wait()
                gather(t + 1, 1 - b2, (q + 1) % 3).start()
        return 0
    lax.fori_loop(0, CHUNKS // 6, six, 0)
    scatter_wait((CHUNKS - 1) % 2, (CHUNKS - 1) % 3)

    plsc.subcore_barrier()
    poff = 0
    while poff < SLABP:
        n = min(K, SLABP - poff)
        pltpu.sync_copy(acc.at[pl.ds(p0 + poff, n)],
                        out.at[cid, pl.ds(p0 + poff, n)])
        poff += n

    @pl.when(sid == 0)
    def _write_tail():
        pltpu.sync_copy(acc.at[pl.ds(NS * SLABP, TAILP)],
                        out.at[cid, pl.ds(NS * SLABP, TAILP)])


def _combine_body(p_ref, o_ref):
    o_ref[0] = jnp.concatenate([p_ref[0].T, p_ref[1].T], axis=0)


_combine = pl.pallas_call(
    _combine_body,
    out_shape=jax.ShapeDtypeStruct((1, C, N_NODES), jnp.float32),
)


def kernel(x, iInd, jInd, W):
    xT = jnp.swapaxes(x[0], 0, 1)
    xP = jnp.stack([xT[:, :CH].reshape(NP, C), xT[:, CH:].reshape(NP, C)])
    pad = E_PAD - iInd.shape[0]
    iP = jnp.concatenate([iInd, jnp.zeros((pad,), jnp.int32)])
    jP = jnp.concatenate([jInd, jnp.zeros((pad,), jnp.int32)])
    wP = jnp.concatenate([W, jnp.zeros((pad,), jnp.float32)])
    iojo = CH * (iP & 1) + ((CH * (jP & 1)) << 8)
    meta = jnp.concatenate([
        (iP >> 1).reshape(NS, CHUNKS, 1, K),
        (jP >> 1).reshape(NS, CHUNKS, 1, K),
        iojo.reshape(NS, CHUNKS, 1, K),
    ], axis=2)
    wgt = wP.reshape(NS, CHUNKS, K)

    sc = pl.kernel(
        _sc_body,
        out_type=jax.ShapeDtypeStruct((NC, NP, C), jnp.float32),
        mesh=plsc.VectorSubcoreMesh(core_axis_name="c", subcore_axis_name="s"),
        scratch_types=[
            pltpu.VMEM_SHARED((NP, C), jnp.float32),
            pltpu.VMEM_SHARED((NP, C), jnp.float32),
            pltpu.VMEM((3, 3, K), jnp.int32),
            pltpu.VMEM((3, K), jnp.float32),
            pltpu.VMEM((2, K, C), jnp.float32),
            pltpu.SemaphoreType.DMA,
            pltpu.SemaphoreType.DMA,
            pltpu.SemaphoreType.DMA,
            pltpu.SemaphoreType.DMA,
            pltpu.SemaphoreType.DMA,
            pltpu.SemaphoreType.DMA,
            pltpu.SemaphoreType.DMA,
            pltpu.SemaphoreType.DMA,
            pltpu.SemaphoreType.DMA,
        ],
    )
    partial = sc(xP, meta, wgt)
    ph = partial.reshape(NC, N_NODES, CH)
    return _combine(ph)


# speedup vs baseline: 1.7509x; 1.7509x over previous
import jax
import jax.numpy as jnp
from jax import lax
from jax.experimental import pallas as pl
from jax.experimental.pallas import tpu as pltpu
from jax.experimental.pallas import tpu_sc as plsc

N_NODES = 10000
C = 128
N_EDGES = 320000

NC = 2
NS = 16
NW = NC * NS
K = 128
CHUNKS = -(-N_EDGES // (NW * K))
PER_W = CHUNKS * K
E_PAD = PER_W * NW
SLAB = 624
TAIL0 = N_NODES - NS * SLAB


def _sc_body(xT, meta, wgt, out, acc, midx, wbuf, rows, msem0, msem1,
             wsem0, wsem1):
    cid = lax.axis_index("c")
    sid = lax.axis_index("s")
    wid = cid * NS + sid
    msems = (msem0, msem1)
    wsems = (wsem0, wsem1)

    def meta_cp(ch, b):
        return pltpu.make_async_copy(meta.at[wid, ch], midx.at[b], msems[b])

    def wgt_cp(ch, b):
        return pltpu.make_async_copy(wgt.at[wid, ch], wbuf.at[b], wsems[b])

    def scale_chunk(b):
        def scale(g8, _):
            wv = wbuf[b, pl.ds(16 * g8, 16)]
            w2v = wv * wv
            for l in range(16):
                e = 16 * g8 + l
                w2 = w2v[l]
                for j in range(C // 16):
                    rows[e, pl.ds(16 * j, 16)] = (
                        rows[e, pl.ds(16 * j, 16)] * w2)
            return 0
        lax.fori_loop(0, K // 16, scale, 0)

    def zero_row(i, _):
        for j in range(C // 16):
            rows[i, pl.ds(16 * j, 16)] = jnp.zeros((16,), jnp.float32)
        return 0
    lax.fori_loop(0, K, zero_row, 0)
    r0 = sid * SLAB
    off = 0
    while off < SLAB:
        n = min(K, SLAB - off)
        pltpu.sync_copy(rows.at[pl.ds(0, n)], acc.at[pl.ds(r0 + off, n)])
        off += n

    @pl.when(sid == 0)
    def _zero_tail():
        pltpu.sync_copy(rows.at[pl.ds(0, TAIL0)],
                        acc.at[pl.ds(NS * SLAB, TAIL0)])
    plsc.subcore_barrier()

    meta_cp(0, 0).start()
    wgt_cp(0, 0).start()
    meta_cp(0, 0).wait()
    wgt_cp(0, 0).wait()

    def pair(g, _):
        for b in range(2):
            ch = 2 * g + b
            b1 = 1 - b

            @pl.when(ch + 1 < CHUNKS)
            def _prefetch():
                meta_cp(ch + 1, b1).start()
                wgt_cp(ch + 1, b1).start()

            pltpu.sync_copy(xT.at[midx.at[b, 1]], rows)
            scale_chunk(b)
            pltpu.sync_copy(rows, acc.at[midx.at[b, 0]], add=True)

            @pl.when(ch + 1 < CHUNKS)
            def _wait_next_meta():
                meta_cp(ch + 1, b1).wait()
                wgt_cp(ch + 1, b1).wait()
        return 0
    lax.fori_loop(0, CHUNKS // 2, pair, 0)

    bl = (CHUNKS - 1) % 2
    pltpu.sync_copy(xT.at[midx.at[bl, 1]], rows)
    scale_chunk(bl)
    pltpu.sync_copy(rows, acc.at[midx.at[bl, 0]], add=True)

    plsc.subcore_barrier()
    pltpu.sync_copy(acc.at[pl.ds(r0, SLAB)], out.at[cid, pl.ds(r0, SLAB)])

    @pl.when(sid == 0)
    def _write_tail():
        pltpu.sync_copy(acc.at[pl.ds(NS * SLAB, TAIL0)],
                        out.at[cid, pl.ds(NS * SLAB, TAIL0)])


def _combine_body(p_ref, o_ref):
    s = p_ref[0] + p_ref[1]
    o_ref[0] = s.T


_combine = pl.pallas_call(
    _combine_body,
    out_shape=jax.ShapeDtypeStruct((1, C, N_NODES), jnp.float32),
)


def kernel(x, iInd, jInd, W):
    xT = jnp.swapaxes(x[0], 0, 1)
    pad = E_PAD - iInd.shape[0]
    iP = jnp.concatenate([iInd, jnp.zeros((pad,), jnp.int32)])
    jP = jnp.concatenate([jInd, jnp.zeros((pad,), jnp.int32)])
    wP = jnp.concatenate([W, jnp.zeros((pad,), jnp.float32)])
    meta = jnp.concatenate([
        iP.reshape(NW, CHUNKS, 1, K),
        jP.reshape(NW, CHUNKS, 1, K),
    ], axis=2)
    wgt = wP.reshape(NW, CHUNKS, K)

    sc = pl.kernel(
        _sc_body,
        out_type=jax.ShapeDtypeStruct((NC, N_NODES, C), jnp.float32),
        mesh=plsc.VectorSubcoreMesh(core_axis_name="c", subcore_axis_name="s"),
        scratch_types=[
            pltpu.VMEM_SHARED((N_NODES, C), jnp.float32),
            pltpu.VMEM((2, 2, K), jnp.int32),
            pltpu.VMEM((2, K), jnp.float32),
            pltpu.VMEM((K, C), jnp.float32),
            pltpu.SemaphoreType.DMA,
            pltpu.SemaphoreType.DMA,
            pltpu.SemaphoreType.DMA,
            pltpu.SemaphoreType.DMA,
        ],
    )
    partial = sc(xT, meta, wgt)
    return _combine(partial)


# speedup vs baseline: 1.9494x; 1.1134x over previous
import jax
import jax.numpy as jnp
from jax import lax
from jax.experimental import pallas as pl
from jax.experimental.pallas import tpu as pltpu
from jax.experimental.pallas import tpu_sc as plsc

N_NODES = 10000
C = 128
N_EDGES = 320000

NC = 2
NS = 16
NW = NC * NS
K = 128
CHUNKS = -(-N_EDGES // (NW * K))
PER_W = CHUNKS * K
E_PAD = PER_W * NW
SLAB = 624
TAIL0 = N_NODES - NS * SLAB


def _sc_body(xT, meta, wgt, out, acc, midx, wbuf, rows, msem0, msem1, msem2,
             wsem0, wsem1, wsem2, ssem):
    cid = lax.axis_index("c")
    sid = lax.axis_index("s")
    wid = cid * NS + sid
    msems = (msem0, msem1, msem2)
    wsems = (wsem0, wsem1, wsem2)

    def meta_cp(ch, b):
        return pltpu.make_async_copy(meta.at[wid, ch], midx.at[b], msems[b])

    def wgt_cp(ch, b):
        return pltpu.make_async_copy(wgt.at[wid, ch], wbuf.at[b], wsems[b])

    def scale_chunk(b2, b3):
        def scale(g8, _):
            wv = wbuf[b3, pl.ds(16 * g8, 16)]
            w2v = wv * wv
            for l in range(16):
                e = 16 * g8 + l
                w2 = w2v[l]
                for j in range(C // 16):
                    rows[b2, e, pl.ds(16 * j, 16)] = (
                        rows[b2, e, pl.ds(16 * j, 16)] * w2)
            return 0
        lax.fori_loop(0, K // 16, scale, 0)

    def scatter_wait(b2, b3):
        pltpu.make_async_copy(rows.at[b2], acc.at[midx.at[b3, 0]], ssem).wait()

    def zero_row(i, _):
        for j in range(C // 16):
            rows[0, i, pl.ds(16 * j, 16)] = jnp.zeros((16,), jnp.float32)
        return 0
    lax.fori_loop(0, K, zero_row, 0)
    r0 = sid * SLAB
    off = 0
    while off < SLAB:
        n = min(K, SLAB - off)
        pltpu.sync_copy(rows.at[0, pl.ds(0, n)], acc.at[pl.ds(r0 + off, n)])
        off += n

    @pl.when(sid == 0)
    def _zero_tail():
        pltpu.sync_copy(rows.at[0, pl.ds(0, TAIL0)],
                        acc.at[pl.ds(NS * SLAB, TAIL0)])
    plsc.subcore_barrier()

    meta_cp(0, 0).start()
    wgt_cp(0, 0).start()
    meta_cp(0, 0).wait()
    wgt_cp(0, 0).wait()

    def six(g, _):
        for q in range(6):
            ch = 6 * g + q
            b2 = q % 2
            b3 = q % 3

            meta_cp(ch + 1, (q + 1) % 3).start()
            wgt_cp(ch + 1, (q + 1) % 3).start()

            pltpu.sync_copy(xT.at[midx.at[b3, 1]], rows.at[b2])
            scale_chunk(b2, b3)

            @pl.when(ch >= 1)
            def _drain():
                scatter_wait(1 - b2, (q + 2) % 3)
            pltpu.async_copy(rows.at[b2], acc.at[midx.at[b3, 0]], ssem,
                             add=True)

            meta_cp(ch + 1, (q + 1) % 3).wait()
            wgt_cp(ch + 1, (q + 1) % 3).wait()
        return 0
    lax.fori_loop(0, (CHUNKS - 1) // 6, six, 0)

    pltpu.sync_copy(xT.at[midx.at[0, 1]], rows.at[0])
    scale_chunk(0, 0)
    scatter_wait(1, 2)
    pltpu.sync_copy(rows.at[0], acc.at[midx.at[0, 0]], add=True)

    plsc.subcore_barrier()
    pltpu.sync_copy(acc.at[pl.ds(r0, SLAB)], out.at[cid, pl.ds(r0, SLAB)])

    @pl.when(sid == 0)
    def _write_tail():
        pltpu.sync_copy(acc.at[pl.ds(NS * SLAB, TAIL0)],
                        out.at[cid, pl.ds(NS * SLAB, TAIL0)])


def _combine_body(p_ref, o_ref):
    s = p_ref[0] + p_ref[1]
    o_ref[0] = s.T


_combine = pl.pallas_call(
    _combine_body,
    out_shape=jax.ShapeDtypeStruct((1, C, N_NODES), jnp.float32),
)


def kernel(x, iInd, jInd, W):
    xT = jnp.swapaxes(x[0], 0, 1)
    pad = E_PAD - iInd.shape[0]
    iP = jnp.concatenate([iInd, jnp.zeros((pad,), jnp.int32)])
    jP = jnp.concatenate([jInd, jnp.zeros((pad,), jnp.int32)])
    wP = jnp.concatenate([W, jnp.zeros((pad,), jnp.float32)])
    meta = jnp.concatenate([
        iP.reshape(NW, CHUNKS, 1, K),
        jP.reshape(NW, CHUNKS, 1, K),
    ], axis=2)
    wgt = wP.reshape(NW, CHUNKS, K)

    sc = pl.kernel(
        _sc_body,
        out_type=jax.ShapeDtypeStruct((NC, N_NODES, C), jnp.float32),
        mesh=plsc.VectorSubcoreMesh(core_axis_name="c", subcore_axis_name="s"),
        scratch_types=[
            pltpu.VMEM_SHARED((N_NODES, C), jnp.float32),
            pltpu.VMEM((3, 2, K), jnp.int32),
            pltpu.VMEM((3, K), jnp.float32),
            pltpu.VMEM((2, K, C), jnp.float32),
            pltpu.SemaphoreType.DMA,
            pltpu.SemaphoreType.DMA,
            pltpu.SemaphoreType.DMA,
            pltpu.SemaphoreType.DMA,
            pltpu.SemaphoreType.DMA,
            pltpu.SemaphoreType.DMA,
            pltpu.SemaphoreType.DMA,
        ],
    )
    partial = sc(xT, meta, wgt)
    return _combine(partial)
